# baseline (speedup 1.0000x reference)
import functools

import jax
import jax.numpy as jnp
from jax import lax
from jax.experimental import pallas as pl
from jax.experimental.pallas import tpu as pltpu
from jax.experimental.pallas import tpu_sc as plsc

N = 10000
E = 320000
D = 128
NCLS = 10

NC = 2
NS = 16
NW = NC * NS
CH = 80
GRP = 5
EPT = E // NW
NG = EPT // (CH * GRP)
NSLOT = 4
PCH = EPT // CH
NROW = 10240
RPT = NROW // NS
ZB = 640
DUMMY = 10016

_mesh = plsc.VectorSubcoreMesh(core_axis_name="c", subcore_axis_name="s")
_sc_params = pltpu.CompilerParams(needs_layout_passes=False, disable_bounds_checks=True)


@functools.partial(
    pl.kernel,
    out_type=jax.ShapeDtypeStruct((NC, NROW), jnp.float32),
    mesh=_mesh,
    compiler_params=_sc_params,
    scratch_types=[
        pltpu.VMEM((ZB,), jnp.float32),
        pltpu.VMEM((CH,), jnp.float32),
        pltpu.VMEM((2, GRP, CH), jnp.int32),
        pltpu.VMEM_SHARED((NROW,), jnp.float32),
        pltpu.SemaphoreType.DMA,
        pltpu.SemaphoreType.DMA,
    ],
)
def _sc_deg(srcp_hbm, deg_hbm,
            zvec_v, ones_v, srcv, deg_sh, isem, ssem):
    c = lax.axis_index("c")
    s = lax.axis_index("s")
    base_t = c * (E // NC) + s * EPT

    zeros16 = jnp.zeros((16,), jnp.float32)
    ones16 = jnp.ones((16,), jnp.float32)

    def zfill(i, _):
        zvec_v[pl.ds(i * 16, 16)] = zeros16
        return 0

    lax.fori_loop(0, ZB // 16, zfill, 0)
    for j in range(CH // 16):
        ones_v[pl.ds(j * 16, 16)] = ones16

    pltpu.sync_copy(zvec_v, deg_sh.at[pl.ds(s * ZB, ZB)])

    def fire_idx(o, slot):
        for b in range(GRP):
            base = base_t + (o * GRP + b) * CH
            pltpu.async_copy(srcp_hbm.at[pl.ds(base, CH)], srcv.at[slot, b], isem)

    def drain_idx(o, slot):
        for b in range(GRP):
            base = base_t + (o * GRP + b) * CH
            pltpu.make_async_copy(srcp_hbm.at[pl.ds(base, CH)], srcv.at[slot, b], isem).wait()

    fire_idx(0, 0)
    plsc.subcore_barrier()

    def body(o, _):
        g = lax.rem(o, 2)
        g1 = lax.rem(o + 1, 2)
        drain_idx(o, g)

        @pl.when(o > 0)
        def _():
            for b in range(GRP):
                pltpu.make_async_copy(ones_v, deg_sh.at[srcv.at[g1, b]], ssem).wait()

        @pl.when(o < NG - 1)
        def _():
            fire_idx(o + 1, g1)

        for b in range(GRP):
            pltpu.async_copy(ones_v, deg_sh.at[srcv.at[g, b]], ssem, add=True)
        return 0

    lax.fori_loop(0, NG, body, 0)
    for b in range(GRP):
        pltpu.make_async_copy(ones_v, deg_sh.at[srcv.at[0, b]], ssem).wait()
    plsc.subcore_barrier()
    pltpu.sync_copy(deg_sh.at[pl.ds(s * ZB, ZB)], deg_hbm.at[c, pl.ds(s * ZB, ZB)])


@functools.partial(
    pl.kernel,
    out_type=jax.ShapeDtypeStruct((NC, NROW, D), jnp.float32),
    mesh=_mesh,
    compiler_params=_sc_params,
    scratch_types=[
        pltpu.VMEM((32, D), jnp.float32),
        pltpu.VMEM((NSLOT, CH), jnp.int32),
        pltpu.VMEM((NSLOT, CH), jnp.int32),
        pltpu.VMEM((NSLOT, CH, D), jnp.float32),
        pltpu.VMEM_SHARED((NROW, D), jnp.float32),
        pltpu.SemaphoreType.DMA,
        pltpu.SemaphoreType.DMA,
        pltpu.SemaphoreType.DMA,
    ],
)
def _sc_prop(src_hbm, dstp_hbm, tab_hbm, acc_hbm,
             zrows_v, srcv, dstv, rows_v, acc_sh, isem, gsem, ssem):
    c = lax.axis_index("c")
    s = lax.axis_index("s")
    wid = c * NS + s
    base_t = wid * EPT

    zeros16 = jnp.zeros((16,), jnp.float32)

    def zfill(i, _):
        for cc in range(8):
            zrows_v[i, pl.ds(cc * 16, 16)] = zeros16
        return 0

    lax.fori_loop(0, 32, zfill, 0)
    for k in range(20):
        pltpu.async_copy(zrows_v, acc_sh.at[pl.ds(s * RPT + k * 32, 32)], ssem)
    for k in range(20):
        pltpu.make_async_copy(zrows_v, acc_sh.at[pl.ds(s * RPT + k * 32, 32)], ssem).wait()

    def fire_idx(o, slot):
        base = base_t + o * CH
        pltpu.async_copy(src_hbm.at[pl.ds(base, CH)], srcv.at[slot], isem)
        pltpu.async_copy(dstp_hbm.at[pl.ds(base, CH)], dstv.at[slot], isem)

    def drain_idx(o, slot):
        base = base_t + o * CH
        pltpu.make_async_copy(src_hbm.at[pl.ds(base, CH)], srcv.at[slot], isem).wait()
        pltpu.make_async_copy(dstp_hbm.at[pl.ds(base, CH)], dstv.at[slot], isem).wait()

    def drain_gather(slot):
        pltpu.make_async_copy(tab_hbm.at[srcv.at[slot]], rows_v.at[slot], gsem).wait()

    def fire_scatter(slot):
        pltpu.async_copy(rows_v.at[slot], acc_sh.at[dstv.at[slot]], ssem, add=True)

    def drain_scatter(slot):
        pltpu.make_async_copy(rows_v.at[slot], acc_sh.at[dstv.at[slot]], ssem).wait()

    fire_idx(0, 0)
    plsc.subcore_barrier()

    def body(o, _):
        g = lax.rem(o, NSLOT)
        gp1 = lax.rem(o + 1, NSLOT)
        gm1 = lax.rem(o + NSLOT - 1, NSLOT)
        gm3 = lax.rem(o + NSLOT - 3, NSLOT)
        drain_idx(o, g)
        pltpu.async_copy(tab_hbm.at[srcv.at[g]], rows_v.at[g], gsem)

        @pl.when(o > 0)
        def _():
            drain_gather(gm1)
            fire_scatter(gm1)

        @pl.when(o > 2)
        def _():
            drain_scatter(gm3)

        @pl.when(o < PCH - 1)
        def _():
            fire_idx(o + 1, gp1)

        return 0

    lax.fori_loop(0, PCH, body, 0)
    s124 = (PCH - 1) % NSLOT
    drain_gather(s124)
    fire_scatter(s124)
    for o in (PCH - 3, PCH - 2, PCH - 1):
        drain_scatter(o % NSLOT)

    plsc.subcore_barrier()
    pltpu.sync_copy(acc_sh.at[pl.ds(s * RPT, RPT)], acc_hbm.at[c, pl.ds(s * RPT, RPT)])


EF = 2500


def _tc_fix_body(src_ref, dst_ref, srcp_ref, dstp_ref):
    sv = src_ref[...]
    dv = dst_ref[...]
    ne = sv != dv
    srcp_ref[...] = jnp.where(ne, sv, DUMMY)
    dstp_ref[...] = jnp.where(ne, dv, DUMMY)


def _tc_fix(src2d, dst2d):
    return pl.pallas_call(
        _tc_fix_body,
        out_shape=[
            jax.ShapeDtypeStruct((EF, 128), jnp.int32),
            jax.ShapeDtypeStruct((EF, 128), jnp.int32),
        ],
    )(src2d, dst2d)


def _tc_dinv_body(deg_ref, dinv_ref):
    deg = deg_ref[0:1, :] + deg_ref[1:2, :]
    r = jnp.where(deg > 0.0, lax.rsqrt(deg), 0.0)
    dinv_ref[...] = r.reshape(NROW, 1)


def _tc_dinv(deg2):
    return pl.pallas_call(
        _tc_dinv_body,
        out_shape=jax.ShapeDtypeStruct((NROW, 1), jnp.float32),
    )(deg2)


_BM = 2000


def _tc_xscale_body(x_ref, dinv_ref, w0_ref, w2_ref, b_ref, xt_ref, xw_ref):
    x = x_ref[...]
    xt_ref[...] = x * dinv_ref[...]
    w02 = w0_ref[...] - w2_ref[...]
    xw_ref[...] = jnp.dot(x, w02, preferred_element_type=jnp.float32) + b_ref[...]


def _tc_xscale(x, dinv_col, W0, W2, bias2d):
    return pl.pallas_call(
        _tc_xscale_body,
        grid=(N // _BM,),
        in_specs=[
            pl.BlockSpec((_BM, D), lambda i: (i, 0)),
            pl.BlockSpec((_BM, 1), lambda i: (i, 0)),
            pl.BlockSpec((D, D), lambda i: (0, 0)),
            pl.BlockSpec((D, D), lambda i: (0, 0)),
            pl.BlockSpec((1, D), lambda i: (0, 0)),
        ],
        out_specs=[
            pl.BlockSpec((_BM, D), lambda i: (i, 0)),
            pl.BlockSpec((_BM, D), lambda i: (i, 0)),
        ],
        out_shape=[
            jax.ShapeDtypeStruct((N, D), jnp.float32),
            jax.ShapeDtypeStruct((N, D), jnp.float32),
        ],
    )(x, dinv_col, W0, W2, bias2d)


def _tc_mid_body(xw_ref, u1_ref, dinv_ref, w1_ref, p1t_ref, out0_ref):
    dv = dinv_ref[...]
    u1 = u1_ref[0] + u1_ref[1]
    p1 = -dv * u1
    p1t_ref[...] = dv * p1
    out0_ref[...] = xw_ref[...] + jnp.dot(p1, w1_ref[...],
                                          preferred_element_type=jnp.float32)


def _tc_mid(xw, u1parts, dinv_col, W1):
    return pl.pallas_call(
        _tc_mid_body,
        grid=(N // _BM,),
        in_specs=[
            pl.BlockSpec((_BM, D), lambda i: (i, 0)),
            pl.BlockSpec((NC, _BM, D), lambda i: (0, i, 0)),
            pl.BlockSpec((_BM, 1), lambda i: (i, 0)),
            pl.BlockSpec((D, D), lambda i: (0, 0)),
        ],
        out_specs=[
            pl.BlockSpec((_BM, D), lambda i: (i, 0)),
            pl.BlockSpec((_BM, D), lambda i: (i, 0)),
        ],
        out_shape=[
            jax.ShapeDtypeStruct((N, D), jnp.float32),
            jax.ShapeDtypeStruct((N, D), jnp.float32),
        ],
    )(xw, u1parts, dinv_col, W1)


def _tc_final_body(out0_ref, u2_ref, dinv_ref, w2_ref, h_ref):
    u2 = u2_ref[0] + u2_ref[1]
    p2 = -dinv_ref[...] * u2
    out = out0_ref[...] + 2.0 * jnp.dot(p2, w2_ref[...],
                                        preferred_element_type=jnp.float32)
    h_ref[...] = jnp.maximum(out, 0.0)


def _tc_final(out0, u2parts, dinv_col, W2):
    return pl.pallas_call(
        _tc_final_body,
        grid=(N // _BM,),
        in_specs=[
            pl.BlockSpec((_BM, D), lambda i: (i, 0)),
            pl.BlockSpec((NC, _BM, D), lambda i: (0, i, 0)),
            pl.BlockSpec((_BM, 1), lambda i: (i, 0)),
            pl.BlockSpec((D, D), lambda i: (0, 0)),
        ],
        out_specs=pl.BlockSpec((_BM, D), lambda i: (i, 0)),
        out_shape=jax.ShapeDtypeStruct((N, D), jnp.float32),
    )(out0, u2parts, dinv_col, W2)


_FB = _BM * D


def _tc_logits_body(h_ref, lw_ref, lb_ref, logits_ref):
    i = pl.program_id(0)
    contrib = jnp.sum(lw_ref[...] * h_ref[...], axis=1)

    @pl.when(i == 0)
    def _():
        logits_ref[...] = lb_ref[...]

    logits_ref[...] += contrib[None, :]


def _tc_logits(hflat, lin_w, lin_b2d):
    return pl.pallas_call(
        _tc_logits_body,
        grid=(N * D // _FB,),
        in_specs=[
            pl.BlockSpec((1, _FB), lambda i: (0, i)),
            pl.BlockSpec((NCLS, _FB), lambda i: (0, i)),
            pl.BlockSpec((1, NCLS), lambda i: (0, 0)),
        ],
        out_specs=pl.BlockSpec((1, NCLS), lambda i: (0, 0)),
        out_shape=jax.ShapeDtypeStruct((1, NCLS), jnp.float32),
    )(hflat, lin_w, lin_b2d)


def kernel(x, edge_index, batch, W0, W1, W2, conv_bias, lin_w, lin_b):
    src = edge_index[0]
    dst = edge_index[1]

    srcp2d, dstp2d = _tc_fix(src.reshape(EF, 128), dst.reshape(EF, 128))
    srcp = srcp2d.reshape(E)
    dstp = dstp2d.reshape(E)
    deg2 = _sc_deg(srcp)
    dinv_col = _tc_dinv(deg2)
    xt, xw = _tc_xscale(x, dinv_col, W0, W2, conv_bias.reshape(1, D))
    u1parts = _sc_prop(src, dstp, xt)
    p1t, out0 = _tc_mid(xw, u1parts, dinv_col, W1)
    u2parts = _sc_prop(src, dstp, p1t)
    h = _tc_final(out0, u2parts, dinv_col, W2)
    logits = _tc_logits(h.reshape(1, N * D), lin_w, lin_b.reshape(1, NCLS))
    return logits


# speedup vs baseline: 1.0038x; 1.0038x over previous
import functools

import jax
import jax.numpy as jnp
from jax import lax
from jax.experimental import pallas as pl
from jax.experimental.pallas import tpu as pltpu
from jax.experimental.pallas import tpu_sc as plsc

N = 10000
E = 320000
D = 128
NCLS = 10

NC = 2
NS = 16
NW = NC * NS
CH = 80
GRP = 5
EPT = E // NW
NG = EPT // (CH * GRP)
NSLOT = 4
PCH = EPT // CH
NROW = 10240
RPT = NROW // NS
ZB = 640
DUMMY = 10016

_mesh = plsc.VectorSubcoreMesh(core_axis_name="c", subcore_axis_name="s")
_sc_params = pltpu.CompilerParams(needs_layout_passes=False, disable_bounds_checks=True)


@functools.partial(
    pl.kernel,
    out_type=jax.ShapeDtypeStruct((NC, NROW), jnp.float32),
    mesh=_mesh,
    compiler_params=_sc_params,
    scratch_types=[
        pltpu.VMEM((ZB,), jnp.float32),
        pltpu.VMEM((CH,), jnp.float32),
        pltpu.VMEM((2, GRP, CH), jnp.int32),
        pltpu.VMEM_SHARED((NROW,), jnp.float32),
        pltpu.SemaphoreType.DMA,
        pltpu.SemaphoreType.DMA,
    ],
)
def _sc_deg(srcp_hbm, deg_hbm,
            zvec_v, ones_v, srcv, deg_sh, isem, ssem):
    c = lax.axis_index("c")
    s = lax.axis_index("s")
    base_t = c * (E // NC) + s * EPT

    zeros16 = jnp.zeros((16,), jnp.float32)
    ones16 = jnp.ones((16,), jnp.float32)

    def zfill(i, _):
        zvec_v[pl.ds(i * 16, 16)] = zeros16
        return 0

    lax.fori_loop(0, ZB // 16, zfill, 0)
    for j in range(CH // 16):
        ones_v[pl.ds(j * 16, 16)] = ones16

    pltpu.sync_copy(zvec_v, deg_sh.at[pl.ds(s * ZB, ZB)])

    def fire_idx(o, slot):
        for b in range(GRP):
            base = base_t + (o * GRP + b) * CH
            pltpu.async_copy(srcp_hbm.at[pl.ds(base, CH)], srcv.at[slot, b], isem)

    def drain_idx(o, slot):
        for b in range(GRP):
            base = base_t + (o * GRP + b) * CH
            pltpu.make_async_copy(srcp_hbm.at[pl.ds(base, CH)], srcv.at[slot, b], isem).wait()

    fire_idx(0, 0)
    plsc.subcore_barrier()

    def body(o, _):
        g = lax.rem(o, 2)
        g1 = lax.rem(o + 1, 2)
        drain_idx(o, g)

        @pl.when(o > 0)
        def _():
            for b in range(GRP):
                pltpu.make_async_copy(ones_v, deg_sh.at[srcv.at[g1, b]], ssem).wait()

        @pl.when(o < NG - 1)
        def _():
            fire_idx(o + 1, g1)

        for b in range(GRP):
            pltpu.async_copy(ones_v, deg_sh.at[srcv.at[g, b]], ssem, add=True)
        return 0

    lax.fori_loop(0, NG, body, 0)
    for b in range(GRP):
        pltpu.make_async_copy(ones_v, deg_sh.at[srcv.at[0, b]], ssem).wait()
    plsc.subcore_barrier()
    pltpu.sync_copy(deg_sh.at[pl.ds(s * ZB, ZB)], deg_hbm.at[c, pl.ds(s * ZB, ZB)])


@functools.partial(
    pl.kernel,
    out_type=jax.ShapeDtypeStruct((NC, NROW, D), jnp.float32),
    mesh=_mesh,
    compiler_params=_sc_params,
    scratch_types=[
        pltpu.VMEM((32, D), jnp.float32),
        pltpu.VMEM((NSLOT, CH), jnp.int32),
        pltpu.VMEM((NSLOT, CH), jnp.int32),
        pltpu.VMEM((NSLOT, CH, D), jnp.float32),
        pltpu.VMEM_SHARED((NROW, D), jnp.float32),
        pltpu.SemaphoreType.DMA,
        pltpu.SemaphoreType.DMA,
        pltpu.SemaphoreType.DMA,
    ],
)
def _sc_prop(src_hbm, dstp_hbm, tab_hbm, acc_hbm,
             zrows_v, srcv, dstv, rows_v, acc_sh, isem, gsem, ssem):
    c = lax.axis_index("c")
    s = lax.axis_index("s")
    wid = c * NS + s
    base_t = wid * EPT

    zeros16 = jnp.zeros((16,), jnp.float32)

    def zfill(i, _):
        for cc in range(8):
            zrows_v[i, pl.ds(cc * 16, 16)] = zeros16
        return 0

    lax.fori_loop(0, 32, zfill, 0)
    for k in range(20):
        pltpu.async_copy(zrows_v, acc_sh.at[pl.ds(s * RPT + k * 32, 32)], ssem)
    for k in range(20):
        pltpu.make_async_copy(zrows_v, acc_sh.at[pl.ds(s * RPT + k * 32, 32)], ssem).wait()

    def fire_idx(o, slot):
        base = base_t + o * CH
        pltpu.async_copy(src_hbm.at[pl.ds(base, CH)], srcv.at[slot], isem)
        pltpu.async_copy(dstp_hbm.at[pl.ds(base, CH)], dstv.at[slot], isem)

    def drain_idx(o, slot):
        base = base_t + o * CH
        pltpu.make_async_copy(src_hbm.at[pl.ds(base, CH)], srcv.at[slot], isem).wait()
        pltpu.make_async_copy(dstp_hbm.at[pl.ds(base, CH)], dstv.at[slot], isem).wait()

    def drain_gather(slot):
        pltpu.make_async_copy(tab_hbm.at[srcv.at[slot]], rows_v.at[slot], gsem).wait()

    def fire_scatter(slot):
        pltpu.async_copy(rows_v.at[slot], acc_sh.at[dstv.at[slot]], ssem, add=True)

    def drain_scatter(slot):
        pltpu.make_async_copy(rows_v.at[slot], acc_sh.at[dstv.at[slot]], ssem).wait()

    fire_idx(0, 0)
    plsc.subcore_barrier()

    def body(o, _):
        g = lax.rem(o, NSLOT)
        gp1 = lax.rem(o + 1, NSLOT)
        gm1 = lax.rem(o + NSLOT - 1, NSLOT)
        gm3 = lax.rem(o + NSLOT - 3, NSLOT)
        drain_idx(o, g)
        pltpu.async_copy(tab_hbm.at[srcv.at[g]], rows_v.at[g], gsem)

        @pl.when(o > 0)
        def _():
            drain_gather(gm1)
            fire_scatter(gm1)

        @pl.when(o > 2)
        def _():
            drain_scatter(gm3)

        @pl.when(o < PCH - 1)
        def _():
            fire_idx(o + 1, gp1)

        return 0

    lax.fori_loop(0, PCH, body, 0)
    s124 = (PCH - 1) % NSLOT
    drain_gather(s124)
    fire_scatter(s124)
    for o in (PCH - 3, PCH - 2, PCH - 1):
        drain_scatter(o % NSLOT)

    plsc.subcore_barrier()
    pltpu.sync_copy(acc_sh.at[pl.ds(s * RPT, RPT)], acc_hbm.at[c, pl.ds(s * RPT, RPT)])


EF = 2500


def _tc_fix_body(src_ref, dst_ref, srcp_ref, dstp_ref):
    sv = src_ref[...]
    dv = dst_ref[...]
    ne = sv != dv
    srcp_ref[...] = jnp.where(ne, sv, DUMMY)
    dstp_ref[...] = jnp.where(ne, dv, DUMMY)


def _tc_fix(src2d, dst2d):
    return pl.pallas_call(
        _tc_fix_body,
        out_shape=[
            jax.ShapeDtypeStruct((EF, 128), jnp.int32),
            jax.ShapeDtypeStruct((EF, 128), jnp.int32),
        ],
    )(src2d, dst2d)


def _tc_dinv_body(deg_ref, dinv_ref):
    deg = deg_ref[0:1, :] + deg_ref[1:2, :]
    r = jnp.where(deg > 0.0, lax.rsqrt(deg), 0.0)
    dinv_ref[...] = r.reshape(NROW, 1)


def _tc_dinv(deg2):
    return pl.pallas_call(
        _tc_dinv_body,
        out_shape=jax.ShapeDtypeStruct((NROW, 1), jnp.float32),
    )(deg2)


_BM = 2000


def _tc_xscale_body(x_ref, dinv_ref, xt_ref):
    xt_ref[...] = x_ref[...] * dinv_ref[...]


def _tc_xscale(x, dinv_col):
    return pl.pallas_call(
        _tc_xscale_body,
        grid=(N // _BM,),
        in_specs=[
            pl.BlockSpec((_BM, D), lambda i: (i, 0)),
            pl.BlockSpec((_BM, 1), lambda i: (i, 0)),
        ],
        out_specs=pl.BlockSpec((_BM, D), lambda i: (i, 0)),
        out_shape=jax.ShapeDtypeStruct((N, D), jnp.float32),
    )(x, dinv_col)


def _tc_xw_body(x_ref, w0_ref, w2_ref, b_ref, xw_ref):
    w02 = w0_ref[...] - w2_ref[...]
    xw_ref[...] = jnp.dot(x_ref[...], w02,
                          preferred_element_type=jnp.float32) + b_ref[...]


def _tc_xw(x, W0, W2, bias2d):
    return pl.pallas_call(
        _tc_xw_body,
        grid=(N // _BM,),
        in_specs=[
            pl.BlockSpec((_BM, D), lambda i: (i, 0)),
            pl.BlockSpec((D, D), lambda i: (0, 0)),
            pl.BlockSpec((D, D), lambda i: (0, 0)),
            pl.BlockSpec((1, D), lambda i: (0, 0)),
        ],
        out_specs=pl.BlockSpec((_BM, D), lambda i: (i, 0)),
        out_shape=jax.ShapeDtypeStruct((N, D), jnp.float32),
    )(x, W0, W2, bias2d)


def _tc_mid_body(xw_ref, u1_ref, dinv_ref, w1_ref, p1t_ref, out0_ref):
    dv = dinv_ref[...]
    u1 = u1_ref[0] + u1_ref[1]
    p1 = -dv * u1
    p1t_ref[...] = dv * p1
    out0_ref[...] = xw_ref[...] + jnp.dot(p1, w1_ref[...],
                                          preferred_element_type=jnp.float32)


def _tc_mid(xw, u1parts, dinv_col, W1):
    return pl.pallas_call(
        _tc_mid_body,
        grid=(N // _BM,),
        in_specs=[
            pl.BlockSpec((_BM, D), lambda i: (i, 0)),
            pl.BlockSpec((NC, _BM, D), lambda i: (0, i, 0)),
            pl.BlockSpec((_BM, 1), lambda i: (i, 0)),
            pl.BlockSpec((D, D), lambda i: (0, 0)),
        ],
        out_specs=[
            pl.BlockSpec((_BM, D), lambda i: (i, 0)),
            pl.BlockSpec((_BM, D), lambda i: (i, 0)),
        ],
        out_shape=[
            jax.ShapeDtypeStruct((N, D), jnp.float32),
            jax.ShapeDtypeStruct((N, D), jnp.float32),
        ],
    )(xw, u1parts, dinv_col, W1)


def _tc_final_body(out0_ref, u2_ref, dinv_ref, w2_ref, h_ref):
    u2 = u2_ref[0] + u2_ref[1]
    p2 = -dinv_ref[...] * u2
    out = out0_ref[...] + 2.0 * jnp.dot(p2, w2_ref[...],
                                        preferred_element_type=jnp.float32)
    h_ref[...] = jnp.maximum(out, 0.0)


def _tc_final(out0, u2parts, dinv_col, W2):
    return pl.pallas_call(
        _tc_final_body,
        grid=(N // _BM,),
        in_specs=[
            pl.BlockSpec((_BM, D), lambda i: (i, 0)),
            pl.BlockSpec((NC, _BM, D), lambda i: (0, i, 0)),
            pl.BlockSpec((_BM, 1), lambda i: (i, 0)),
            pl.BlockSpec((D, D), lambda i: (0, 0)),
        ],
        out_specs=pl.BlockSpec((_BM, D), lambda i: (i, 0)),
        out_shape=jax.ShapeDtypeStruct((N, D), jnp.float32),
    )(out0, u2parts, dinv_col, W2)


_FB = _BM * D


def _tc_logits_body(h_ref, lw_ref, lb_ref, logits_ref):
    i = pl.program_id(0)
    contrib = jnp.sum(lw_ref[...] * h_ref[...], axis=1)

    @pl.when(i == 0)
    def _():
        logits_ref[...] = lb_ref[...]

    logits_ref[...] += contrib[None, :]


def _tc_logits(hflat, lin_w, lin_b2d):
    return pl.pallas_call(
        _tc_logits_body,
        grid=(N * D // _FB,),
        in_specs=[
            pl.BlockSpec((1, _FB), lambda i: (0, i)),
            pl.BlockSpec((NCLS, _FB), lambda i: (0, i)),
            pl.BlockSpec((1, NCLS), lambda i: (0, 0)),
        ],
        out_specs=pl.BlockSpec((1, NCLS), lambda i: (0, 0)),
        out_shape=jax.ShapeDtypeStruct((1, NCLS), jnp.float32),
    )(hflat, lin_w, lin_b2d)


def kernel(x, edge_index, batch, W0, W1, W2, conv_bias, lin_w, lin_b):
    src = edge_index[0]
    dst = edge_index[1]

    srcp2d, dstp2d = _tc_fix(src.reshape(EF, 128), dst.reshape(EF, 128))
    srcp = srcp2d.reshape(E)
    dstp = dstp2d.reshape(E)
    deg2 = _sc_deg(srcp)
    dinv_col = _tc_dinv(deg2)
    xt = _tc_xscale(x, dinv_col)
    xw = _tc_xw(x, W0, W2, conv_bias.reshape(1, D))
    u1parts = _sc_prop(src, dstp, xt)
    p1t, out0 = _tc_mid(xw, u1parts, dinv_col, W1)
    u2parts = _sc_prop(src, dstp, p1t)
    h = _tc_final(out0, u2parts, dinv_col, W2)
    logits = _tc_logits(h.reshape(1, N * D), lin_w, lin_b.reshape(1, NCLS))
    return logits
